# Initial kernel scaffold; baseline (speedup 1.0000x reference)
import jax
import jax.numpy as jnp
from jax.experimental import pallas as pl


def kernel(x, labels, centers):
    raise NotImplementedError("write your pallas kernel here")


# speedup vs baseline: 1.0572x; 1.0572x over previous
import functools

import jax
import jax.numpy as jnp
from jax import lax
from jax.experimental import pallas as pl
from jax.experimental.pallas import tpu as pltpu
from jax.experimental.pallas import tpu_sc as plsc

_NC = 2
_NS = 16
_L = 16
_NW = _NC * _NS


@functools.lru_cache(maxsize=None)
def _make_sc_kernel(B, D, chunk):
    b_per_w = B // _NW
    n_chunks = b_per_w // chunk
    mesh = plsc.VectorSubcoreMesh(core_axis_name="c", subcore_axis_name="s")

    @functools.partial(
        pl.kernel,
        mesh=mesh,
        out_type=jax.ShapeDtypeStruct((_NW, _L), jnp.float32),
        scratch_types=[
            pltpu.VMEM((chunk,), jnp.int32),
            pltpu.VMEM((chunk, D), jnp.float32),
            pltpu.VMEM((chunk, D), jnp.float32),
            pltpu.VMEM((_L,), jnp.float32),
            pltpu.SemaphoreType.DMA,
        ],
        compiler_params=pltpu.CompilerParams(needs_layout_passes=False),
    )
    def sc_kernel(x_hbm, lab_hbm, cen_hbm, out_hbm, idx_v, x_v, c_v, acc_v, sem):
        wid = lax.axis_index("s") * _NC + lax.axis_index("c")
        base = wid * b_per_w
        tot = jnp.float32(0.0)
        for ci in range(n_chunks):
            row0 = base + ci * chunk
            pltpu.sync_copy(lab_hbm.at[pl.ds(row0, chunk)], idx_v)
            gather = pltpu.async_copy(cen_hbm.at[idx_v], c_v, sem)
            pltpu.sync_copy(x_hbm.at[pl.ds(row0, chunk)], x_v)
            gather.wait()

            def row_body(r, tot):
                acc = jnp.zeros((_L,), jnp.float32)
                for s in range(D // _L):
                    d = x_v[r, pl.ds(s * _L, _L)] - c_v[r, pl.ds(s * _L, _L)]
                    acc = acc + d * d
                dist = jnp.sum(acc)
                dist = jnp.minimum(jnp.maximum(dist, 1e-12), 1e12)
                return tot + dist

            tot = lax.fori_loop(0, chunk, row_body, tot)
        lane = lax.iota(jnp.int32, _L)
        acc_v[...] = jnp.where(lane == 0, tot * (1.0 / B), 0.0)
        pltpu.sync_copy(acc_v, out_hbm.at[wid])

    return sc_kernel


def kernel(x, labels, centers):
    B, D = x.shape
    partials = _make_sc_kernel(B, D, 256)(x, labels.astype(jnp.int32), centers)
    return jnp.sum(partials)


# speedup vs baseline: 1.1533x; 1.0909x over previous
import functools

import jax
import jax.numpy as jnp
from jax import lax
from jax.experimental import pallas as pl
from jax.experimental.pallas import tpu as pltpu
from jax.experimental.pallas import tpu_sc as plsc

_NC = 2
_NS = 16
_L = 16
_NW = _NC * _NS


@functools.lru_cache(maxsize=None)
def _make_sc_kernel(B, D, chunk):
    b_per_w = B // _NW
    n_chunks = b_per_w // chunk
    mesh = plsc.VectorSubcoreMesh(core_axis_name="c", subcore_axis_name="s")

    @functools.partial(
        pl.kernel,
        mesh=mesh,
        out_type=jax.ShapeDtypeStruct((_NW, _L), jnp.float32),
        scratch_types=[
            pltpu.VMEM((b_per_w,), jnp.int32),
            pltpu.VMEM((chunk, D), jnp.float32),
            pltpu.VMEM((chunk, D), jnp.float32),
            pltpu.VMEM((chunk, D), jnp.float32),
            pltpu.VMEM((chunk, D), jnp.float32),
            pltpu.VMEM((_L,), jnp.float32),
            pltpu.SemaphoreType.DMA,
            pltpu.SemaphoreType.DMA,
            pltpu.SemaphoreType.DMA,
            pltpu.SemaphoreType.DMA,
        ],
        compiler_params=pltpu.CompilerParams(needs_layout_passes=False),
    )
    def sc_kernel(x_hbm, lab_hbm, cen_hbm, out_hbm,
                  idx_v, x_v0, c_v0, x_v1, c_v1, acc_v,
                  sx0, sc0, sx1, sc1, ):
        wid = lax.axis_index("s") * _NC + lax.axis_index("c")
        base = wid * b_per_w
        bufs = ((x_v0, c_v0, sx0, sc0), (x_v1, c_v1, sx1, sc1))

        pltpu.sync_copy(lab_hbm.at[pl.ds(base, b_per_w)], idx_v)

        def start(ci):
            xb, cb, sx, sc = bufs[ci % 2]
            row0 = base + ci * chunk
            hx = pltpu.async_copy(x_hbm.at[pl.ds(row0, chunk)], xb, sx)
            hc = pltpu.async_copy(
                cen_hbm.at[idx_v.at[pl.ds(ci * chunk, chunk)]], cb, sc)
            return hx, hc

        pending = start(0)
        tot = jnp.float32(0.0)
        for ci in range(n_chunks):
            nxt = start(ci + 1) if ci + 1 < n_chunks else None
            pending[0].wait()
            pending[1].wait()
            xb, cb, _, _ = bufs[ci % 2]

            def row_body(r, tot):
                acc = jnp.zeros((_L,), jnp.float32)
                for s in range(D // _L):
                    d = xb[r, pl.ds(s * _L, _L)] - cb[r, pl.ds(s * _L, _L)]
                    acc = acc + d * d
                dist = jnp.sum(acc)
                dist = jnp.minimum(jnp.maximum(dist, 1e-12), 1e12)
                return tot + dist

            tot = lax.fori_loop(0, chunk, row_body, tot)
            pending = nxt
        lane = lax.iota(jnp.int32, _L)
        acc_v[...] = jnp.where(lane == 0, tot * (1.0 / B), 0.0)
        pltpu.sync_copy(acc_v, out_hbm.at[wid])

    return sc_kernel


def kernel(x, labels, centers):
    B, D = x.shape
    partials = _make_sc_kernel(B, D, 128)(x, labels.astype(jnp.int32), centers)
    return jnp.sum(partials)


# speedup vs baseline: 1.1662x; 1.0112x over previous
import functools

import jax
import jax.numpy as jnp
from jax import lax
from jax.experimental import pallas as pl
from jax.experimental.pallas import tpu as pltpu
from jax.experimental.pallas import tpu_sc as plsc

_NC = 2
_NS = 16
_L = 16
_NW = _NC * _NS


@functools.lru_cache(maxsize=None)
def _make_sc_kernel(B, D, chunk):
    b_per_w = B // _NW
    n_chunks = b_per_w // chunk
    mesh = plsc.VectorSubcoreMesh(core_axis_name="c", subcore_axis_name="s")

    @functools.partial(
        pl.kernel,
        mesh=mesh,
        out_type=jax.ShapeDtypeStruct((_NW, _L), jnp.float32),
        scratch_types=[
            pltpu.VMEM((b_per_w,), jnp.int32),
            pltpu.VMEM((b_per_w, D), jnp.float32),
            pltpu.VMEM((chunk, D), jnp.float32),
            pltpu.VMEM((chunk, D), jnp.float32),
            pltpu.VMEM((_L,), jnp.float32),
            pltpu.SemaphoreType.DMA,
            pltpu.SemaphoreType.DMA,
            pltpu.SemaphoreType.DMA,
        ],
        compiler_params=pltpu.CompilerParams(needs_layout_passes=False),
    )
    def sc_kernel(x_hbm, lab_hbm, cen_hbm, out_hbm,
                  idx_v, x_v, c_v0, c_v1, acc_v,
                  sx, sc0, sc1):
        wid = lax.axis_index("s") * _NC + lax.axis_index("c")
        base = wid * b_per_w
        cbufs = ((c_v0, sc0), (c_v1, sc1))

        hx = pltpu.async_copy(x_hbm.at[pl.ds(base, b_per_w)], x_v, sx)
        pltpu.sync_copy(lab_hbm.at[pl.ds(base, b_per_w)], idx_v)

        def start(ci):
            cb, sem = cbufs[ci % 2]
            return pltpu.async_copy(
                cen_hbm.at[idx_v.at[pl.ds(ci * chunk, chunk)]], cb, sem)

        pending = start(0)
        hx.wait()
        tot = jnp.float32(0.0)
        for ci in range(n_chunks):
            nxt = start(ci + 1) if ci + 1 < n_chunks else None
            pending.wait()
            cb, _ = cbufs[ci % 2]
            r0 = ci * chunk

            def row_body(r, tot):
                acc = jnp.zeros((_L,), jnp.float32)
                for s in range(D // _L):
                    d = x_v[r0 + r, pl.ds(s * _L, _L)] - cb[r, pl.ds(s * _L, _L)]
                    acc = acc + d * d
                dist = jnp.sum(acc)
                dist = jnp.minimum(jnp.maximum(dist, 1e-12), 1e12)
                return tot + dist

            tot = lax.fori_loop(0, chunk, row_body, tot)
            pending = nxt
        lane = lax.iota(jnp.int32, _L)
        acc_v[...] = jnp.where(lane == 0, tot * (1.0 / B), 0.0)
        pltpu.sync_copy(acc_v, out_hbm.at[wid])

    return sc_kernel


def kernel(x, labels, centers):
    B, D = x.shape
    partials = _make_sc_kernel(B, D, 128)(x, labels.astype(jnp.int32), centers)
    return jnp.sum(partials)
